# baseline (speedup 1.0000x reference)
import functools

import jax
import jax.numpy as jnp
from jax import lax
from jax.experimental import pallas as pl
from jax.experimental.pallas import tpu as pltpu
from jax.experimental.pallas import tpu_sc as plsc

NC = 2
NS = 16
L = 16
NW = NC * NS

N = 32768
D = 1024

M_TC = 4096
N_SC = N - M_TC
TPW = N_SC // NW
K = 32
G = TPW // K
LOOP_CHUNKS = 3 * ((G - 2) // 3)

BLK = 512


@functools.partial(
    pl.kernel,
    out_type=(
        jax.ShapeDtypeStruct((N_SC, D), jnp.float32),
        jax.ShapeDtypeStruct((N_SC, D), jnp.float32),
    ),
    mesh=plsc.VectorSubcoreMesh(core_axis_name="c", subcore_axis_name="s"),
    scratch_types=[
        pltpu.VMEM((TPW,), jnp.int32),
        pltpu.VMEM((TPW,), jnp.float32),
        pltpu.VMEM((TPW,), jnp.float32),
        pltpu.VMEM((3, K, D), jnp.float32),
        pltpu.SemaphoreType.DMA,
        pltpu.SemaphoreType.DMA,
        pltpu.SemaphoreType.DMA,
        pltpu.SemaphoreType.DMA,
        pltpu.SemaphoreType.DMA,
        pltpu.SemaphoreType.DMA,
        pltpu.SemaphoreType.DMA,
        pltpu.SemaphoreType.DMA,
        pltpu.SemaphoreType.DMA,
    ],
)
def _select_rows_sc(x_hbm, mask_hbm, out1_hbm, out2_hbm, idx_v, m0_v, m1_v,
                    rows_v, sem_in0, sem_in1, sem_in2,
                    sem_a0, sem_a1, sem_a2, sem_b0, sem_b1, sem_b2):
    wid = lax.axis_index("s") * NC + lax.axis_index("c")
    base = wid * TPW
    tok0 = M_TC + base

    pltpu.sync_copy(mask_hbm.at[pl.ds(tok0, TPW)], m0_v)
    pltpu.sync_copy(mask_hbm.at[pl.ds(N + tok0, TPW)], m1_v)

    def compute_idx(g):
        for j in range(K // L):
            o = g * K + j * L
            m0 = m0_v[pl.ds(o, L)]
            m1 = m1_v[pl.ds(o, L)]
            sel = jnp.where(m0 >= m1, jnp.zeros((L,), jnp.int32),
                            jnp.full((L,), N, jnp.int32))
            idx_v[pl.ds(o, L)] = tok0 + o + lax.iota(jnp.int32, L) + sel

    sem_in = (sem_in0, sem_in1, sem_in2)
    sem_a = (sem_a0, sem_a1, sem_a2)
    sem_b = (sem_b0, sem_b1, sem_b2)

    def gather_dma(g, s):
        return pltpu.make_async_copy(
            x_hbm.at[idx_v.at[pl.ds(g * K, K)]], rows_v.at[s], sem_in[s])

    def store_dma(g, s, out_hbm, sem):
        return pltpu.make_async_copy(
            rows_v.at[s], out_hbm.at[pl.ds(base + g * K, K)], sem[s])

    def chunk_step(g, s):
        so = (s + 2) % 3

        gather_dma(g, s).wait()
        store_dma(g, s, out1_hbm, sem_a).start()
        store_dma(g, s, out2_hbm, sem_b).start()

        @pl.when(g + 2 < G)
        def _():
            compute_idx(g + 2)

            @pl.when(g >= 1)
            def _():
                store_dma(g - 1, so, out1_hbm, sem_a).wait()
                store_dma(g - 1, so, out2_hbm, sem_b).wait()

            gather_dma(g + 2, so).start()

    compute_idx(0)
    compute_idx(1)
    gather_dma(0, 0).start()
    gather_dma(1, 1).start()

    def outer(t, carry):
        chunk_step(3 * t, 0)
        chunk_step(3 * t + 1, 1)
        chunk_step(3 * t + 2, 2)
        return carry

    lax.fori_loop(0, LOOP_CHUNKS // 3, outer, 0)
    for g in range(LOOP_CHUNKS, G):
        chunk_step(g, g % 3)
    for g in range(G - 3, G):
        store_dma(g, g % 3, out1_hbm, sem_a).wait()
        store_dma(g, g % 3, out2_hbm, sem_b).wait()


def _select_tc_body(x0_ref, x1_ref, m0_ref, m1_ref, o1_ref, o2_ref):
    picked = jnp.where(m0_ref[...] >= m1_ref[...], x0_ref[...], x1_ref[...])
    o1_ref[...] = picked
    o2_ref[...] = picked


_select_rows_tc = pl.pallas_call(
    _select_tc_body,
    grid=(M_TC // BLK,),
    in_specs=[
        pl.BlockSpec((BLK, D), lambda i: (i, 0)),
        pl.BlockSpec((BLK, D), lambda i: (N // BLK + i, 0)),
        pl.BlockSpec((BLK, 1), lambda i: (i, 0)),
        pl.BlockSpec((BLK, 1), lambda i: (i, 0)),
    ],
    out_specs=[
        pl.BlockSpec((BLK, D), lambda i: (i, 0)),
        pl.BlockSpec((BLK, D), lambda i: (i, 0)),
    ],
    out_shape=[
        jax.ShapeDtypeStruct((M_TC, D), jnp.float32),
        jax.ShapeDtypeStruct((M_TC, D), jnp.float32),
    ],
)


def kernel(x, mask, mask_threshold):
    del mask_threshold
    x_flat = x.reshape(2 * N, D)
    mask_flat = mask.reshape(2 * N)
    m0_col = mask_flat[:N].reshape(N, 1)
    m1_col = mask_flat[N:].reshape(N, 1)

    sc_o1, sc_o2 = _select_rows_sc(x_flat, mask_flat)
    tc_o1, tc_o2 = _select_rows_tc(x_flat, x_flat, m0_col, m1_col)

    o1 = jnp.concatenate([tc_o1, sc_o1], axis=0).reshape(4, 8192, D)
    o2 = jnp.concatenate([tc_o2, sc_o2], axis=0).reshape(4, 8192, D)
    return (o1, o2)


# speedup vs baseline: 2.0263x; 2.0263x over previous
import functools

import jax
import jax.numpy as jnp
from jax import lax
from jax.experimental import pallas as pl
from jax.experimental.pallas import tpu as pltpu
from jax.experimental.pallas import tpu_sc as plsc

NC = 2
NS = 16
L = 16
NW = NC * NS

N = 32768
D = 1024
TPW = N // NW
K = 16
G = TPW // K
LOOP_CHUNKS = 3 * ((G - 2) // 3)


@functools.partial(
    pl.kernel,
    out_type=(
        jax.ShapeDtypeStruct((N, D), jnp.float32),
        jax.ShapeDtypeStruct((N, D), jnp.float32),
    ),
    mesh=plsc.VectorSubcoreMesh(core_axis_name="c", subcore_axis_name="s"),
    scratch_types=[
        pltpu.VMEM((TPW,), jnp.int32),
        pltpu.VMEM((TPW,), jnp.float32),
        pltpu.VMEM((TPW,), jnp.float32),
        pltpu.VMEM((3, K, D), jnp.float32),
        pltpu.SemaphoreType.DMA,
        pltpu.SemaphoreType.DMA,
        pltpu.SemaphoreType.DMA,
        pltpu.SemaphoreType.DMA,
        pltpu.SemaphoreType.DMA,
        pltpu.SemaphoreType.DMA,
        pltpu.SemaphoreType.DMA,
        pltpu.SemaphoreType.DMA,
        pltpu.SemaphoreType.DMA,
    ],
)
def _select_rows(x_hbm, mask_hbm, out1_hbm, out2_hbm, idx_v, m0_v, m1_v,
                 rows_v, sem_in0, sem_in1, sem_in2,
                 sem_a0, sem_a1, sem_a2, sem_b0, sem_b1, sem_b2):
    wid = lax.axis_index("s") * NC + lax.axis_index("c")
    base = wid * TPW

    pltpu.sync_copy(mask_hbm.at[pl.ds(base, TPW)], m0_v)
    pltpu.sync_copy(mask_hbm.at[pl.ds(N + base, TPW)], m1_v)

    def compute_idx(g):
        for j in range(K // L):
            o = g * K + j * L
            m0 = m0_v[pl.ds(o, L)]
            m1 = m1_v[pl.ds(o, L)]
            sel = jnp.where(m0 >= m1, jnp.zeros((L,), jnp.int32),
                            jnp.full((L,), N, jnp.int32))
            idx_v[pl.ds(o, L)] = base + o + lax.iota(jnp.int32, L) + sel

    sem_in = (sem_in0, sem_in1, sem_in2)
    sem_a = (sem_a0, sem_a1, sem_a2)
    sem_b = (sem_b0, sem_b1, sem_b2)

    def gather_dma(g, s):
        return pltpu.make_async_copy(
            x_hbm.at[idx_v.at[pl.ds(g * K, K)]], rows_v.at[s], sem_in[s])

    def store_dma(g, s, out_hbm, sem):
        return pltpu.make_async_copy(
            rows_v.at[s], out_hbm.at[pl.ds(base + g * K, K)], sem[s])

    def chunk_step(g, s):
        so = (s + 2) % 3

        gather_dma(g, s).wait()
        store_dma(g, s, out1_hbm, sem_a).start()
        store_dma(g, s, out2_hbm, sem_b).start()

        @pl.when(g + 2 < G)
        def _():
            compute_idx(g + 2)

            @pl.when(g >= 1)
            def _():
                store_dma(g - 1, so, out1_hbm, sem_a).wait()
                store_dma(g - 1, so, out2_hbm, sem_b).wait()

            gather_dma(g + 2, so).start()

    compute_idx(0)
    compute_idx(1)
    gather_dma(0, 0).start()
    gather_dma(1, 1).start()

    def outer(t, carry):
        chunk_step(3 * t, 0)
        chunk_step(3 * t + 1, 1)
        chunk_step(3 * t + 2, 2)
        return carry

    lax.fori_loop(0, LOOP_CHUNKS // 3, outer, 0)
    for g in range(LOOP_CHUNKS, G):
        chunk_step(g, g % 3)
    for g in range(G - 3, G):
        store_dma(g, g % 3, out1_hbm, sem_a).wait()
        store_dma(g, g % 3, out2_hbm, sem_b).wait()


def kernel(x, mask, mask_threshold):
    del mask_threshold
    x_flat = x.reshape(2 * N, D)
    mask_flat = mask.reshape(2 * N)
    o1, o2 = _select_rows(x_flat, mask_flat)
    return (o1.reshape(4, 8192, D), o2.reshape(4, 8192, D))


# speedup vs baseline: 2.0688x; 1.0210x over previous
import functools

import jax
import jax.numpy as jnp
from jax import lax
from jax.experimental import pallas as pl
from jax.experimental.pallas import tpu as pltpu
from jax.experimental.pallas import tpu_sc as plsc

NC = 2
NS = 16
L = 16
NW = NC * NS

N = 32768
D = 1024
TPW = N // NW
K = 32
G = TPW // K
LOOP_CHUNKS = 3 * ((G - 2) // 3)


@functools.partial(
    pl.kernel,
    out_type=(
        jax.ShapeDtypeStruct((N, D), jnp.float32),
        jax.ShapeDtypeStruct((N, D), jnp.float32),
    ),
    mesh=plsc.VectorSubcoreMesh(core_axis_name="c", subcore_axis_name="s"),
    scratch_types=[
        pltpu.VMEM((TPW,), jnp.int32),
        pltpu.VMEM((TPW,), jnp.float32),
        pltpu.VMEM((TPW,), jnp.float32),
        pltpu.VMEM((3, K, D), jnp.float32),
        pltpu.SemaphoreType.DMA,
        pltpu.SemaphoreType.DMA,
        pltpu.SemaphoreType.DMA,
        pltpu.SemaphoreType.DMA,
        pltpu.SemaphoreType.DMA,
        pltpu.SemaphoreType.DMA,
        pltpu.SemaphoreType.DMA,
        pltpu.SemaphoreType.DMA,
        pltpu.SemaphoreType.DMA,
    ],
)
def _select_rows(x_hbm, mask_hbm, out1_hbm, out2_hbm, idx_v, m0_v, m1_v,
                 rows_v, sem_in0, sem_in1, sem_in2,
                 sem_a0, sem_a1, sem_a2, sem_b0, sem_b1, sem_b2):
    wid = lax.axis_index("s") * NC + lax.axis_index("c")
    base = wid * TPW

    pltpu.sync_copy(mask_hbm.at[pl.ds(base, TPW)], m0_v)
    pltpu.sync_copy(mask_hbm.at[pl.ds(N + base, TPW)], m1_v)

    def compute_idx(g):
        for j in range(K // L):
            o = g * K + j * L
            m0 = m0_v[pl.ds(o, L)]
            m1 = m1_v[pl.ds(o, L)]
            sel = jnp.where(m0 >= m1, jnp.zeros((L,), jnp.int32),
                            jnp.full((L,), N, jnp.int32))
            idx_v[pl.ds(o, L)] = base + o + lax.iota(jnp.int32, L) + sel

    sem_in = (sem_in0, sem_in1, sem_in2)
    sem_a = (sem_a0, sem_a1, sem_a2)
    sem_b = (sem_b0, sem_b1, sem_b2)

    def gather_dma(g, s):
        return pltpu.make_async_copy(
            x_hbm.at[idx_v.at[pl.ds(g * K, K)]], rows_v.at[s], sem_in[s])

    def store_dma(g, s, out_hbm, sem):
        return pltpu.make_async_copy(
            rows_v.at[s], out_hbm.at[pl.ds(base + g * K, K)], sem[s])

    def chunk_step(g, s):
        so = (s + 2) % 3

        gather_dma(g, s).wait()
        store_dma(g, s, out1_hbm, sem_a).start()

        @pl.when(g + 2 < G)
        def _():
            compute_idx(g + 2)

            @pl.when(g >= 1)
            def _():
                store_dma(g - 1, so, out1_hbm, sem_a).wait()
                store_dma(g - 1, so, out2_hbm, sem_b).wait()

            gather_dma(g + 2, so).start()

        store_dma(g, s, out2_hbm, sem_b).start()

    compute_idx(0)
    compute_idx(1)
    gather_dma(0, 0).start()
    gather_dma(1, 1).start()

    def outer(t, carry):
        chunk_step(3 * t, 0)
        chunk_step(3 * t + 1, 1)
        chunk_step(3 * t + 2, 2)
        return carry

    lax.fori_loop(0, LOOP_CHUNKS // 3, outer, 0)
    for g in range(LOOP_CHUNKS, G):
        chunk_step(g, g % 3)
    for g in range(G - 3, G):
        store_dma(g, g % 3, out1_hbm, sem_a).wait()
        store_dma(g, g % 3, out2_hbm, sem_b).wait()


def kernel(x, mask, mask_threshold):
    del mask_threshold
    x_flat = x.reshape(2 * N, D)
    mask_flat = mask.reshape(2 * N)
    o1, o2 = _select_rows(x_flat, mask_flat)
    return (o1.reshape(4, 8192, D), o2.reshape(4, 8192, D))


# speedup vs baseline: 2.0761x; 1.0035x over previous
import functools

import jax
import jax.numpy as jnp
from jax import lax
from jax.experimental import pallas as pl
from jax.experimental.pallas import tpu as pltpu
from jax.experimental.pallas import tpu_sc as plsc

NC = 2
NS = 16
L = 16
NW = NC * NS

N = 32768
D = 1024
TPW = N // NW
K = 32
G = TPW // K
LOOP_CHUNKS = 3 * ((G - 2) // 3)


@functools.partial(
    pl.kernel,
    out_type=(
        jax.ShapeDtypeStruct((N, D), jnp.float32),
        jax.ShapeDtypeStruct((N, D), jnp.float32),
    ),
    mesh=plsc.VectorSubcoreMesh(core_axis_name="c", subcore_axis_name="s"),
    scratch_types=[
        pltpu.VMEM((TPW,), jnp.int32),
        pltpu.VMEM((TPW,), jnp.float32),
        pltpu.VMEM((TPW,), jnp.float32),
        pltpu.VMEM((3, K, D), jnp.float32),
        pltpu.SemaphoreType.DMA,
        pltpu.SemaphoreType.DMA,
        pltpu.SemaphoreType.DMA,
        pltpu.SemaphoreType.DMA,
        pltpu.SemaphoreType.DMA,
        pltpu.SemaphoreType.DMA,
        pltpu.SemaphoreType.DMA,
        pltpu.SemaphoreType.DMA,
        pltpu.SemaphoreType.DMA,
    ],
)
def _select_rows(x_hbm, mask_hbm, out1_hbm, out2_hbm, idx_v, m0_v, m1_v,
                 rows_v, sem_in0, sem_in1, sem_in2,
                 sem_a0, sem_a1, sem_a2, sem_b0, sem_b1, sem_b2):
    wid = lax.axis_index("s") * NC + lax.axis_index("c")
    base = wid * TPW

    m0_copy = pltpu.make_async_copy(mask_hbm.at[pl.ds(base, TPW)], m0_v,
                                    sem_a0)
    m1_copy = pltpu.make_async_copy(mask_hbm.at[pl.ds(N + base, TPW)], m1_v,
                                    sem_b0)
    m0_copy.start()
    m1_copy.start()
    m0_copy.wait()
    m1_copy.wait()

    def compute_idx(g):
        for j in range(K // L):
            o = g * K + j * L
            m0 = m0_v[pl.ds(o, L)]
            m1 = m1_v[pl.ds(o, L)]
            sel = jnp.where(m0 >= m1, jnp.zeros((L,), jnp.int32),
                            jnp.full((L,), N, jnp.int32))
            idx_v[pl.ds(o, L)] = base + o + lax.iota(jnp.int32, L) + sel

    sem_in = (sem_in0, sem_in1, sem_in2)
    sem_a = (sem_a0, sem_a1, sem_a2)
    sem_b = (sem_b0, sem_b1, sem_b2)

    def gather_dma(g, s):
        return pltpu.make_async_copy(
            x_hbm.at[idx_v.at[pl.ds(g * K, K)]], rows_v.at[s], sem_in[s])

    def store_dma(g, s, out_hbm, sem):
        return pltpu.make_async_copy(
            rows_v.at[s], out_hbm.at[pl.ds(base + g * K, K)], sem[s])

    def chunk_step(g, s):
        so = (s + 2) % 3

        gather_dma(g, s).wait()
        store_dma(g, s, out1_hbm, sem_a).start()

        @pl.when(g + 2 < G)
        def _():
            compute_idx(g + 2)

            @pl.when(g >= 1)
            def _():
                store_dma(g - 1, so, out1_hbm, sem_a).wait()
                store_dma(g - 1, so, out2_hbm, sem_b).wait()

            gather_dma(g + 2, so).start()

        store_dma(g, s, out2_hbm, sem_b).start()

    compute_idx(0)
    compute_idx(1)
    gather_dma(0, 0).start()
    gather_dma(1, 1).start()

    def outer(t, carry):
        chunk_step(3 * t, 0)
        chunk_step(3 * t + 1, 1)
        chunk_step(3 * t + 2, 2)
        return carry

    lax.fori_loop(0, LOOP_CHUNKS // 3, outer, 0)
    for g in range(LOOP_CHUNKS, G):
        chunk_step(g, g % 3)
    for g in range(G - 3, G):
        store_dma(g, g % 3, out1_hbm, sem_a).wait()
        store_dma(g, g % 3, out2_hbm, sem_b).wait()


def kernel(x, mask, mask_threshold):
    del mask_threshold
    x_flat = x.reshape(2 * N, D)
    mask_flat = mask.reshape(2 * N)
    o1, o2 = _select_rows(x_flat, mask_flat)
    return (o1.reshape(4, 8192, D), o2.reshape(4, 8192, D))
